# Initial kernel scaffold; baseline (speedup 1.0000x reference)
import functools

import jax
import jax.numpy as jnp
from jax import lax
from jax.experimental import pallas as pl
from jax.experimental.pallas import tpu as pltpu
from jax.experimental.pallas import tpu_sc as plsc

B, N1, N2, C = 8, 4096, 1024, 128
EPS = 1e-5
BLK = 512
NB = N1 // BLK
NBLK = B * NB
CHQ = 128
NW = 32
BLK_PER_W = NBLK // NW


def _k1_body(x1_ref, x2_ref, gidx_ref, wq_ref):
    b = pl.program_id(0)
    d = None
    for c in range(3):
        t = x1_ref[0, c, :][:, None] - x2_ref[0, c, :][None, :]
        d = t * t if d is None else d + t * t
    it = lax.broadcasted_iota(jnp.int32, (BLK, N2), 1)
    inf = jnp.float32(jnp.inf)
    ws = []
    for k in range(3):
        m = jnp.min(d, axis=1, keepdims=True)
        ik = jnp.min(jnp.where(d == m, it, jnp.int32(2**30)), axis=1)
        gidx_ref[0, k, :] = ik + b * N2
        ws.append(1.0 / m[:, 0])
        if k < 2:
            d = jnp.where(it == ik[:, None], inf, d)
    s = (ws[0] + ws[1]) + ws[2]
    for k in range(3):
        wq_ref[0, k, :] = ws[k] / s


def _k1(xyz1t, xyz2t):
    return pl.pallas_call(
        _k1_body,
        grid=(B, NB),
        in_specs=[
            pl.BlockSpec((1, 3, BLK), lambda b, j: (b, 0, j)),
            pl.BlockSpec((1, 3, N2), lambda b, j: (b, 0, 0)),
        ],
        out_specs=[
            pl.BlockSpec((1, 3, BLK), lambda b, j: (b * NB + j, 0, 0)),
            pl.BlockSpec((1, 3, BLK), lambda b, j: (b * NB + j, 0, 0)),
        ],
        out_shape=[
            jax.ShapeDtypeStruct((NBLK, 3, BLK), jnp.int32),
            jax.ShapeDtypeStruct((NBLK, 3, BLK), jnp.float32),
        ],
    )(xyz1t, xyz2t)


def _sc_body(table_hbm, gidx_hbm, wq_hbm, fused_hbm,
             gidx_v, w_v, rows_v, out_v, sem):
    wid = lax.axis_index("s") * 2 + lax.axis_index("c")
    for half in range(BLK_PER_W):
        blk = wid * BLK_PER_W + half
        pltpu.sync_copy(gidx_hbm.at[blk], gidx_v)
        pltpu.sync_copy(wq_hbm.at[blk], w_v)
        for t in range(BLK // CHQ):
            cps = [
                pltpu.async_copy(
                    table_hbm.at[gidx_v.at[k, pl.ds(t * CHQ, CHQ)]],
                    rows_v.at[k], sem)
                for k in range(3)
            ]
            for cp in cps:
                cp.wait()

            def body(q, _):
                qg = t * CHQ + q
                iq = jnp.full((16,), qg, jnp.int32)
                wv = [
                    plsc.load_gather(w_v, [jnp.full((16,), k, jnp.int32), iq])
                    for k in range(3)
                ]
                for cb in range(C // 16):
                    sl = pl.ds(cb * 16, 16)
                    acc = (wv[0] * rows_v[0, q, sl]
                           + wv[1] * rows_v[1, q, sl]) + wv[2] * rows_v[2, q, sl]
                    out_v[q, sl] = acc
                return 0

            lax.fori_loop(0, CHQ, body, 0)
            pltpu.sync_copy(out_v, fused_hbm.at[blk, pl.ds(t * CHQ, CHQ)])


def _sc_gather(table, gidx, wq):
    kern = pl.kernel(
        _sc_body,
        out_type=jax.ShapeDtypeStruct((NBLK, BLK, C), jnp.float32),
        mesh=plsc.VectorSubcoreMesh(core_axis_name="c", subcore_axis_name="s"),
        scratch_types=[
            pltpu.VMEM((3, BLK), jnp.int32),
            pltpu.VMEM((3, BLK), jnp.float32),
            pltpu.VMEM((3, CHQ, C), jnp.float32),
            pltpu.VMEM((CHQ, C), jnp.float32),
            pltpu.SemaphoreType.DMA,
        ],
    )
    return kern(table, gidx, wq)


def _k2_body(p1_ref, f_ref, pb_ref, w_ref, b_ref, h_ref, s_ref, q_ref):
    w = w_ref[...]
    h = lax.dot_general(w[:, :C], p1_ref[0], (((1,), (0,)), ((), ())),
                        preferred_element_type=jnp.float32)
    h = h + lax.dot_general(w[:, C:2 * C], f_ref[0], (((1,), (1,)), ((), ())),
                            preferred_element_type=jnp.float32)
    h = h + lax.dot_general(w[:, 2 * C:], pb_ref[0], (((1,), (0,)), ((), ())),
                            preferred_element_type=jnp.float32)
    h = h + b_ref[...]
    h_ref[0] = h
    s_ref[0] = jnp.sum(h, axis=1, keepdims=True)
    q_ref[0] = jnp.sum(h * h, axis=1, keepdims=True)


def _k2(points1, fused, points_b1, W0, b0c):
    co = W0.shape[0]
    return pl.pallas_call(
        _k2_body,
        grid=(B, NB),
        in_specs=[
            pl.BlockSpec((1, C, BLK), lambda b, j: (b, 0, j)),
            pl.BlockSpec((1, BLK, C), lambda b, j: (b, j, 0)),
            pl.BlockSpec((1, C, BLK), lambda b, j: (b, 0, j)),
            pl.BlockSpec((co, 3 * C), lambda b, j: (0, 0)),
            pl.BlockSpec((co, 1), lambda b, j: (0, 0)),
        ],
        out_specs=[
            pl.BlockSpec((1, co, BLK), lambda b, j: (b, 0, j)),
            pl.BlockSpec((1, co, 1), lambda b, j: (b * NB + j, 0, 0)),
            pl.BlockSpec((1, co, 1), lambda b, j: (b * NB + j, 0, 0)),
        ],
        out_shape=[
            jax.ShapeDtypeStruct((B, co, N1), jnp.float32),
            jax.ShapeDtypeStruct((NBLK, co, 1), jnp.float32),
            jax.ShapeDtypeStruct((NBLK, co, 1), jnp.float32),
        ],
    )(points1, fused, points_b1, W0, b0c)


def _k3_body(h0_ref, a_ref, c_ref, w_ref, b_ref, h_ref, s_ref, q_ref):
    xh = jnp.maximum(h0_ref[0] * a_ref[...] + c_ref[...], 0.0)
    h = lax.dot_general(w_ref[...], xh, (((1,), (0,)), ((), ())),
                        preferred_element_type=jnp.float32)
    h = h + b_ref[...]
    h_ref[0] = h
    s_ref[0] = jnp.sum(h, axis=1, keepdims=True)
    q_ref[0] = jnp.sum(h * h, axis=1, keepdims=True)


def _k3(h0, a0, c0, W1, b1c):
    ci, co = W1.shape[1], W1.shape[0]
    return pl.pallas_call(
        _k3_body,
        grid=(B, NB),
        in_specs=[
            pl.BlockSpec((1, ci, BLK), lambda b, j: (b, 0, j)),
            pl.BlockSpec((ci, 1), lambda b, j: (0, 0)),
            pl.BlockSpec((ci, 1), lambda b, j: (0, 0)),
            pl.BlockSpec((co, ci), lambda b, j: (0, 0)),
            pl.BlockSpec((co, 1), lambda b, j: (0, 0)),
        ],
        out_specs=[
            pl.BlockSpec((1, co, BLK), lambda b, j: (b, 0, j)),
            pl.BlockSpec((1, co, 1), lambda b, j: (b * NB + j, 0, 0)),
            pl.BlockSpec((1, co, 1), lambda b, j: (b * NB + j, 0, 0)),
        ],
        out_shape=[
            jax.ShapeDtypeStruct((B, co, N1), jnp.float32),
            jax.ShapeDtypeStruct((NBLK, co, 1), jnp.float32),
            jax.ShapeDtypeStruct((NBLK, co, 1), jnp.float32),
        ],
    )(h0, a0, c0, W1, b1c)


def _k4_body(h1_ref, a_ref, c_ref, o_ref):
    y = h1_ref[0] * a_ref[...] + c_ref[...]
    o_ref[0, 0, :] = jnp.maximum(jnp.max(y, axis=0), 0.0)


def _k4(h1, a1, c1):
    ci = h1.shape[1]
    return pl.pallas_call(
        _k4_body,
        grid=(B, NB),
        in_specs=[
            pl.BlockSpec((1, ci, BLK), lambda b, j: (b, 0, j)),
            pl.BlockSpec((ci, 1), lambda b, j: (0, 0)),
            pl.BlockSpec((ci, 1), lambda b, j: (0, 0)),
        ],
        out_specs=pl.BlockSpec((1, 1, BLK), lambda b, j: (b, 0, j)),
        out_shape=jax.ShapeDtypeStruct((B, 1, N1), jnp.float32),
    )(h1, a1, c1)


def _bn_coeffs(s, q, gamma, beta):
    n = B * N1
    mean = jnp.sum(s, axis=0) / n
    var = jnp.sum(q, axis=0) / n - mean * mean
    a = gamma[:, None] / jnp.sqrt(var + EPS)
    c = beta[:, None] - mean * a
    return a, c


def kernel(xyz1, xyz2, points2, points1, points_b1,
           W0, b0, gamma0, beta0, W1, b1, gamma1, beta1):
    xyz1t = jnp.transpose(xyz1, (0, 2, 1))
    xyz2t = jnp.transpose(xyz2, (0, 2, 1))
    table = jnp.transpose(points2, (0, 2, 1)).reshape(B * N2, C)

    gidx, wq = _k1(xyz1t, xyz2t)
    fused = _sc_gather(table, gidx, wq).reshape(B, N1, C)

    h0, s0, q0 = _k2(points1, fused, points_b1, W0, b0[:, None])
    a0, c0 = _bn_coeffs(s0, q0, gamma0, beta0)
    h1, s1, q1 = _k3(h0, a0, c0, W1, b1[:, None])
    a1, c1 = _bn_coeffs(s1, q1, gamma1, beta1)
    out = _k4(h1, a1, c1)
    return out.reshape(B, N1)


# speedup vs baseline: 13.5211x; 13.5211x over previous
import functools

import jax
import jax.numpy as jnp
from jax import lax
from jax.experimental import pallas as pl
from jax.experimental.pallas import tpu as pltpu
from jax.experimental.pallas import tpu_sc as plsc

B, N1, N2, C = 8, 4096, 1024, 128
EPS = 1e-5
BLK = 512
NB = N1 // BLK
NBLK = B * NB
CHQ = 128
NW = 32
BLK_PER_W = NBLK // NW


def _k1_body(x1_ref, x2_ref, gidx_ref, wq_ref):
    b = pl.program_id(0)
    d = None
    for c in range(3):
        t = x1_ref[0, c, :][:, None] - x2_ref[0, c, :][None, :]
        d = t * t if d is None else d + t * t
    it = lax.broadcasted_iota(jnp.int32, (BLK, N2), 1)
    inf = jnp.float32(jnp.inf)
    ws = []
    for k in range(3):
        m = jnp.min(d, axis=1, keepdims=True)
        ik = jnp.min(jnp.where(d == m, it, jnp.int32(2**30)), axis=1)
        gidx_ref[0, 0, pl.ds(k * BLK, BLK)] = ik + b * N2
        ws.append(1.0 / m[:, 0])
        if k < 2:
            d = jnp.where(it == ik[:, None], inf, d)
    s = (ws[0] + ws[1]) + ws[2]
    for k in range(3):
        wq_ref[0, 0, pl.ds(k * BLK, BLK)] = ws[k] / s


def _k1(xyz1t, xyz2t):
    return pl.pallas_call(
        _k1_body,
        grid=(B, NB),
        in_specs=[
            pl.BlockSpec((1, 3, BLK), lambda b, j: (b, 0, j)),
            pl.BlockSpec((1, 3, N2), lambda b, j: (b, 0, 0)),
        ],
        out_specs=[
            pl.BlockSpec((1, 1, 3 * BLK), lambda b, j: (b * NB + j, 0, 0)),
            pl.BlockSpec((1, 1, 3 * BLK), lambda b, j: (b * NB + j, 0, 0)),
        ],
        out_shape=[
            jax.ShapeDtypeStruct((NBLK, 1, 3 * BLK), jnp.int32),
            jax.ShapeDtypeStruct((NBLK, 1, 3 * BLK), jnp.float32),
        ],
    )(xyz1t, xyz2t)


def _sc_body(table_hbm, gidx_hbm, gath_hbm, gidx_v, rows0, rows1, rows2, sem):
    wid = lax.axis_index("s") * 2 + lax.axis_index("c")
    rows = (rows0, rows1, rows2)
    for half in range(BLK_PER_W):
        blk = wid * BLK_PER_W + half
        pltpu.sync_copy(gidx_hbm.at[blk], gidx_v)
        for t in range(BLK // CHQ):
            cps = [
                pltpu.async_copy(
                    table_hbm.at[gidx_v.at[pl.ds(k * BLK + t * CHQ, CHQ)]],
                    rows[k], sem)
                for k in range(3)
            ]
            for cp in cps:
                cp.wait()
            for k in range(3):
                pltpu.sync_copy(
                    rows[k], gath_hbm.at[blk, k, pl.ds(t * CHQ, CHQ)])


def _sc_gather(table, gidx):
    kern = pl.kernel(
        _sc_body,
        out_type=jax.ShapeDtypeStruct((NBLK, 3, BLK, C), jnp.float32),
        mesh=plsc.VectorSubcoreMesh(core_axis_name="c", subcore_axis_name="s",
                                    num_cores=2, num_subcores=16),
        scratch_types=[
            pltpu.VMEM((3 * BLK,), jnp.int32),
            pltpu.VMEM((CHQ, C), jnp.float32),
            pltpu.VMEM((CHQ, C), jnp.float32),
            pltpu.VMEM((CHQ, C), jnp.float32),
            pltpu.SemaphoreType.DMA,
        ],
    )
    return kern(table, gidx.reshape(NBLK, 3 * BLK))


def _k2_body(p1_ref, g_ref, wq_ref, pb_ref, w_ref, b_ref, h_ref, s_ref, q_ref):
    w0 = wq_ref[0, 0, pl.ds(0, BLK)][:, None]
    w1 = wq_ref[0, 0, pl.ds(BLK, BLK)][:, None]
    w2 = wq_ref[0, 0, pl.ds(2 * BLK, BLK)][:, None]
    fused = (w0 * g_ref[0, 0] + w1 * g_ref[0, 1]) + w2 * g_ref[0, 2]
    w = w_ref[...]
    h = lax.dot_general(w[:, :C], p1_ref[0], (((1,), (0,)), ((), ())),
                        preferred_element_type=jnp.float32)
    h = h + lax.dot_general(w[:, C:2 * C], fused, (((1,), (1,)), ((), ())),
                            preferred_element_type=jnp.float32)
    h = h + lax.dot_general(w[:, 2 * C:], pb_ref[0], (((1,), (0,)), ((), ())),
                            preferred_element_type=jnp.float32)
    h = h + b_ref[...]
    h_ref[0] = h
    s_ref[0] = jnp.sum(h, axis=1, keepdims=True)
    q_ref[0] = jnp.sum(h * h, axis=1, keepdims=True)


def _k2(points1, gath, wq, points_b1, W0, b0c):
    co = W0.shape[0]
    return pl.pallas_call(
        _k2_body,
        grid=(B, NB),
        in_specs=[
            pl.BlockSpec((1, C, BLK), lambda b, j: (b, 0, j)),
            pl.BlockSpec((1, 3, BLK, C), lambda b, j: (b * NB + j, 0, 0, 0)),
            pl.BlockSpec((1, 1, 3 * BLK), lambda b, j: (b * NB + j, 0, 0)),
            pl.BlockSpec((1, C, BLK), lambda b, j: (b, 0, j)),
            pl.BlockSpec((co, 3 * C), lambda b, j: (0, 0)),
            pl.BlockSpec((co, 1), lambda b, j: (0, 0)),
        ],
        out_specs=[
            pl.BlockSpec((1, co, BLK), lambda b, j: (b, 0, j)),
            pl.BlockSpec((1, co, 1), lambda b, j: (b * NB + j, 0, 0)),
            pl.BlockSpec((1, co, 1), lambda b, j: (b * NB + j, 0, 0)),
        ],
        out_shape=[
            jax.ShapeDtypeStruct((B, co, N1), jnp.float32),
            jax.ShapeDtypeStruct((NBLK, co, 1), jnp.float32),
            jax.ShapeDtypeStruct((NBLK, co, 1), jnp.float32),
        ],
    )(points1, gath, wq, points_b1, W0, b0c)


def _k3_body(h0_ref, a_ref, c_ref, w_ref, b_ref, h_ref, s_ref, q_ref):
    xh = jnp.maximum(h0_ref[0] * a_ref[...] + c_ref[...], 0.0)
    h = lax.dot_general(w_ref[...], xh, (((1,), (0,)), ((), ())),
                        preferred_element_type=jnp.float32)
    h = h + b_ref[...]
    h_ref[0] = h
    s_ref[0] = jnp.sum(h, axis=1, keepdims=True)
    q_ref[0] = jnp.sum(h * h, axis=1, keepdims=True)


def _k3(h0, a0, c0, W1, b1c):
    ci, co = W1.shape[1], W1.shape[0]
    return pl.pallas_call(
        _k3_body,
        grid=(B, NB),
        in_specs=[
            pl.BlockSpec((1, ci, BLK), lambda b, j: (b, 0, j)),
            pl.BlockSpec((ci, 1), lambda b, j: (0, 0)),
            pl.BlockSpec((ci, 1), lambda b, j: (0, 0)),
            pl.BlockSpec((co, ci), lambda b, j: (0, 0)),
            pl.BlockSpec((co, 1), lambda b, j: (0, 0)),
        ],
        out_specs=[
            pl.BlockSpec((1, co, BLK), lambda b, j: (b, 0, j)),
            pl.BlockSpec((1, co, 1), lambda b, j: (b * NB + j, 0, 0)),
            pl.BlockSpec((1, co, 1), lambda b, j: (b * NB + j, 0, 0)),
        ],
        out_shape=[
            jax.ShapeDtypeStruct((B, co, N1), jnp.float32),
            jax.ShapeDtypeStruct((NBLK, co, 1), jnp.float32),
            jax.ShapeDtypeStruct((NBLK, co, 1), jnp.float32),
        ],
    )(h0, a0, c0, W1, b1c)


def _k4_body(h1_ref, a_ref, c_ref, o_ref):
    y = h1_ref[0] * a_ref[...] + c_ref[...]
    o_ref[0, 0, :] = jnp.maximum(jnp.max(y, axis=0), 0.0)


def _k4(h1, a1, c1):
    ci = h1.shape[1]
    return pl.pallas_call(
        _k4_body,
        grid=(B, NB),
        in_specs=[
            pl.BlockSpec((1, ci, BLK), lambda b, j: (b, 0, j)),
            pl.BlockSpec((ci, 1), lambda b, j: (0, 0)),
            pl.BlockSpec((ci, 1), lambda b, j: (0, 0)),
        ],
        out_specs=pl.BlockSpec((1, 1, BLK), lambda b, j: (b, 0, j)),
        out_shape=jax.ShapeDtypeStruct((B, 1, N1), jnp.float32),
    )(h1, a1, c1)


def _bn_coeffs(s, q, gamma, beta):
    n = B * N1
    mean = jnp.sum(s, axis=0) / n
    var = jnp.sum(q, axis=0) / n - mean * mean
    a = gamma[:, None] / jnp.sqrt(var + EPS)
    c = beta[:, None] - mean * a
    return a, c


def kernel(xyz1, xyz2, points2, points1, points_b1,
           W0, b0, gamma0, beta0, W1, b1, gamma1, beta1):
    xyz1t = jnp.transpose(xyz1, (0, 2, 1))
    xyz2t = jnp.transpose(xyz2, (0, 2, 1))
    table = jnp.transpose(points2, (0, 2, 1)).reshape(B * N2, C)

    gidx, wq = _k1(xyz1t, xyz2t)
    gath = _sc_gather(table, gidx)

    h0, s0, q0 = _k2(points1, gath, wq, points_b1, W0, b0[:, None])
    a0, c0 = _bn_coeffs(s0, q0, gamma0, beta0)
    h1, s1, q1 = _k3(h0, a0, c0, W1, b1[:, None])
    a1, c1 = _bn_coeffs(s1, q1, gamma1, beta1)
    out = _k4(h1, a1, c1)
    return out.reshape(B, N1)


# speedup vs baseline: 15.5987x; 1.1537x over previous
import functools

import jax
import jax.numpy as jnp
from jax import lax
from jax.experimental import pallas as pl
from jax.experimental.pallas import tpu as pltpu
from jax.experimental.pallas import tpu_sc as plsc

B, N1, N2, C = 8, 4096, 1024, 128
EPS = 1e-5
BLK = 512
NB = N1 // BLK
NBLK = B * NB
CHQ = 128
NW = 32
BLK_PER_W = NBLK // NW


def _k1_body(x1_ref, x2_ref, gidx_ref, wq_ref):
    b = pl.program_id(0)
    d = None
    for c in range(3):
        t = x1_ref[0, c, :][:, None] - x2_ref[0, c, :][None, :]
        d = t * t if d is None else d + t * t
    it2 = lax.broadcasted_iota(jnp.int32, (N2, 2), 0)
    itcols = jnp.where(lax.broadcasted_iota(jnp.int32, (N2, 2), 1) == 0,
                       it2 >> 5, it2 & 31).astype(jnp.float32)
    inf = jnp.float32(jnp.inf)
    ws = []
    for k in range(3):
        m = jnp.min(d, axis=1, keepdims=True)
        onef = jnp.where(d == m, 1.0, 0.0)
        ikf = lax.dot_general(onef, itcols, (((1,), (0,)), ((), ())),
                              preferred_element_type=jnp.float32)
        iki = ikf.astype(jnp.int32)
        ik = jnp.minimum((iki[:, 0:1] << 5) + iki[:, 1:2], N2 - 1)
        gidx_ref[0, pl.ds(k * BLK, BLK), :] = ik + b * N2
        ws.append(1.0 / m)
        if k < 2:
            d = jnp.where(d == m, inf, d)
    s = (ws[0] + ws[1]) + ws[2]
    for k in range(3):
        wq_ref[0, pl.ds(k * BLK, BLK), :] = ws[k] / s


def _k1(xyz1t, xyz2t):
    return pl.pallas_call(
        _k1_body,
        grid=(B, NB),
        in_specs=[
            pl.BlockSpec((1, 3, BLK), lambda b, j: (b, 0, j)),
            pl.BlockSpec((1, 3, N2), lambda b, j: (b, 0, 0)),
        ],
        out_specs=[
            pl.BlockSpec((1, 3 * BLK, 1), lambda b, j: (b * NB + j, 0, 0)),
            pl.BlockSpec((1, 3 * BLK, 1), lambda b, j: (b * NB + j, 0, 0)),
        ],
        out_shape=[
            jax.ShapeDtypeStruct((NBLK, 3 * BLK, 1), jnp.int32),
            jax.ShapeDtypeStruct((NBLK, 3 * BLK, 1), jnp.float32),
        ],
    )(xyz1t, xyz2t)


def _sc_body(table_hbm, gidx_hbm, gath_hbm, gidx_v, rows0, rows1, rows2, sem):
    wid = lax.axis_index("s") * 2 + lax.axis_index("c")
    rows = (rows0, rows1, rows2)
    for half in range(BLK_PER_W):
        blk = wid * BLK_PER_W + half
        pltpu.sync_copy(gidx_hbm.at[blk], gidx_v)
        for t in range(BLK // CHQ):
            cps = [
                pltpu.async_copy(
                    table_hbm.at[gidx_v.at[pl.ds(k * BLK + t * CHQ, CHQ)]],
                    rows[k], sem)
                for k in range(3)
            ]
            for cp in cps:
                cp.wait()
            for k in range(3):
                pltpu.sync_copy(
                    rows[k], gath_hbm.at[blk, k, pl.ds(t * CHQ, CHQ)])


def _sc_gather(table, gidx):
    kern = pl.kernel(
        _sc_body,
        out_type=jax.ShapeDtypeStruct((NBLK, 3, BLK, C), jnp.float32),
        mesh=plsc.VectorSubcoreMesh(core_axis_name="c", subcore_axis_name="s",
                                    num_cores=2, num_subcores=16),
        scratch_types=[
            pltpu.VMEM((3 * BLK,), jnp.int32),
            pltpu.VMEM((CHQ, C), jnp.float32),
            pltpu.VMEM((CHQ, C), jnp.float32),
            pltpu.VMEM((CHQ, C), jnp.float32),
            pltpu.SemaphoreType.DMA,
        ],
    )
    return kern(table, gidx.reshape(NBLK, 3 * BLK))


def _k2_body(p1_ref, g_ref, wq_ref, pb_ref, w_ref, b_ref, h_ref, s_ref, q_ref):
    w0 = wq_ref[0, pl.ds(0, BLK), :]
    w1 = wq_ref[0, pl.ds(BLK, BLK), :]
    w2 = wq_ref[0, pl.ds(2 * BLK, BLK), :]
    fused = (w0 * g_ref[0, 0] + w1 * g_ref[0, 1]) + w2 * g_ref[0, 2]
    w = w_ref[...]
    h = lax.dot_general(w[:, :C], p1_ref[0], (((1,), (0,)), ((), ())),
                        preferred_element_type=jnp.float32)
    h = h + lax.dot_general(w[:, C:2 * C], fused, (((1,), (1,)), ((), ())),
                            preferred_element_type=jnp.float32)
    h = h + lax.dot_general(w[:, 2 * C:], pb_ref[0], (((1,), (0,)), ((), ())),
                            preferred_element_type=jnp.float32)
    h = h + b_ref[...]
    h_ref[0] = h
    s_ref[0] = jnp.sum(h, axis=1, keepdims=True)
    q_ref[0] = jnp.sum(h * h, axis=1, keepdims=True)


def _k2(points1, gath, wq, points_b1, W0, b0c):
    co = W0.shape[0]
    return pl.pallas_call(
        _k2_body,
        grid=(B, NB),
        in_specs=[
            pl.BlockSpec((1, C, BLK), lambda b, j: (b, 0, j)),
            pl.BlockSpec((1, 3, BLK, C), lambda b, j: (b * NB + j, 0, 0, 0)),
            pl.BlockSpec((1, 3 * BLK, 1), lambda b, j: (b * NB + j, 0, 0)),
            pl.BlockSpec((1, C, BLK), lambda b, j: (b, 0, j)),
            pl.BlockSpec((co, 3 * C), lambda b, j: (0, 0)),
            pl.BlockSpec((co, 1), lambda b, j: (0, 0)),
        ],
        out_specs=[
            pl.BlockSpec((1, co, BLK), lambda b, j: (b, 0, j)),
            pl.BlockSpec((1, co, 1), lambda b, j: (b * NB + j, 0, 0)),
            pl.BlockSpec((1, co, 1), lambda b, j: (b * NB + j, 0, 0)),
        ],
        out_shape=[
            jax.ShapeDtypeStruct((B, co, N1), jnp.float32),
            jax.ShapeDtypeStruct((NBLK, co, 1), jnp.float32),
            jax.ShapeDtypeStruct((NBLK, co, 1), jnp.float32),
        ],
    )(points1, gath, wq, points_b1, W0, b0c)


def _k3_body(h0_ref, a_ref, c_ref, w_ref, b_ref, h_ref, s_ref, q_ref):
    xh = jnp.maximum(h0_ref[0] * a_ref[...] + c_ref[...], 0.0)
    h = lax.dot_general(w_ref[...], xh, (((1,), (0,)), ((), ())),
                        preferred_element_type=jnp.float32)
    h = h + b_ref[...]
    h_ref[0] = h
    s_ref[0] = jnp.sum(h, axis=1, keepdims=True)
    q_ref[0] = jnp.sum(h * h, axis=1, keepdims=True)


def _k3(h0, a0, c0, W1, b1c):
    ci, co = W1.shape[1], W1.shape[0]
    return pl.pallas_call(
        _k3_body,
        grid=(B, NB),
        in_specs=[
            pl.BlockSpec((1, ci, BLK), lambda b, j: (b, 0, j)),
            pl.BlockSpec((ci, 1), lambda b, j: (0, 0)),
            pl.BlockSpec((ci, 1), lambda b, j: (0, 0)),
            pl.BlockSpec((co, ci), lambda b, j: (0, 0)),
            pl.BlockSpec((co, 1), lambda b, j: (0, 0)),
        ],
        out_specs=[
            pl.BlockSpec((1, co, BLK), lambda b, j: (b, 0, j)),
            pl.BlockSpec((1, co, 1), lambda b, j: (b * NB + j, 0, 0)),
            pl.BlockSpec((1, co, 1), lambda b, j: (b * NB + j, 0, 0)),
        ],
        out_shape=[
            jax.ShapeDtypeStruct((B, co, N1), jnp.float32),
            jax.ShapeDtypeStruct((NBLK, co, 1), jnp.float32),
            jax.ShapeDtypeStruct((NBLK, co, 1), jnp.float32),
        ],
    )(h0, a0, c0, W1, b1c)


def _k4_body(h1_ref, a_ref, c_ref, o_ref):
    y = h1_ref[0] * a_ref[...] + c_ref[...]
    o_ref[0, 0, :] = jnp.maximum(jnp.max(y, axis=0), 0.0)


def _k4(h1, a1, c1):
    ci = h1.shape[1]
    return pl.pallas_call(
        _k4_body,
        grid=(B, NB),
        in_specs=[
            pl.BlockSpec((1, ci, BLK), lambda b, j: (b, 0, j)),
            pl.BlockSpec((ci, 1), lambda b, j: (0, 0)),
            pl.BlockSpec((ci, 1), lambda b, j: (0, 0)),
        ],
        out_specs=pl.BlockSpec((1, 1, BLK), lambda b, j: (b, 0, j)),
        out_shape=jax.ShapeDtypeStruct((B, 1, N1), jnp.float32),
    )(h1, a1, c1)


def _bn_coeffs(s, q, gamma, beta):
    n = B * N1
    mean = jnp.sum(s, axis=0) / n
    var = jnp.sum(q, axis=0) / n - mean * mean
    a = gamma[:, None] / jnp.sqrt(var + EPS)
    c = beta[:, None] - mean * a
    return a, c


def kernel(xyz1, xyz2, points2, points1, points_b1,
           W0, b0, gamma0, beta0, W1, b1, gamma1, beta1):
    xyz1t = jnp.transpose(xyz1, (0, 2, 1))
    xyz2t = jnp.transpose(xyz2, (0, 2, 1))
    table = jnp.transpose(points2, (0, 2, 1)).reshape(B * N2, C)

    gidx, wq = _k1(xyz1t, xyz2t)
    gath = _sc_gather(table, gidx)

    h0, s0, q0 = _k2(points1, gath, wq, points_b1, W0, b0[:, None])
    a0, c0 = _bn_coeffs(s0, q0, gamma0, beta0)
    h1, s1, q1 = _k3(h0, a0, c0, W1, b1[:, None])
    a1, c1 = _bn_coeffs(s1, q1, gamma1, beta1)
    out = _k4(h1, a1, c1)
    return out.reshape(B, N1)


# speedup vs baseline: 16.2905x; 1.0443x over previous
import functools

import jax
import jax.numpy as jnp
from jax import lax
from jax.experimental import pallas as pl
from jax.experimental.pallas import tpu as pltpu
from jax.experimental.pallas import tpu_sc as plsc

B, N1, N2, C = 8, 4096, 1024, 128
EPS = 1e-5
BLK = 512
NB = N1 // BLK
NBLK = B * NB
CHQ = 128
NW = 32
BLK_PER_W = NBLK // NW
BLKA = 1024
NBA = N1 // BLKA
BLKB = 2048
NBB = N1 // BLKB
NTOT = B * N1


def _k1_body(x1_ref, x2_ref, gidx_ref, wq_ref):
    b = pl.program_id(0)
    d = None
    for c in range(3):
        t = x1_ref[0, c, :][:, None] - x2_ref[0, c, :][None, :]
        d = t * t if d is None else d + t * t
    it2 = lax.broadcasted_iota(jnp.int32, (N2, 2), 0)
    itcols = jnp.where(lax.broadcasted_iota(jnp.int32, (N2, 2), 1) == 0,
                       it2 >> 5, it2 & 31).astype(jnp.float32)
    inf = jnp.float32(jnp.inf)
    ws = []
    for k in range(3):
        m = jnp.min(d, axis=1, keepdims=True)
        onef = jnp.where(d == m, 1.0, 0.0)
        ikf = lax.dot_general(onef, itcols, (((1,), (0,)), ((), ())),
                              preferred_element_type=jnp.float32)
        iki = ikf.astype(jnp.int32)
        ik = jnp.minimum((iki[:, 0:1] << 5) + iki[:, 1:2], N2 - 1)
        gidx_ref[0, pl.ds(k * BLK, BLK), :] = ik + b * N2
        ws.append(1.0 / m)
        if k < 2:
            d = jnp.where(d == m, inf, d)
    s = (ws[0] + ws[1]) + ws[2]
    for k in range(3):
        wq_ref[0, pl.ds(k * BLK, BLK), :] = jnp.broadcast_to(
            ws[k] / s, (BLK, 16))


def _k1(xyz1t, xyz2t):
    return pl.pallas_call(
        _k1_body,
        grid=(B, NB),
        in_specs=[
            pl.BlockSpec((1, 3, BLK), lambda b, j: (b, 0, j)),
            pl.BlockSpec((1, 3, N2), lambda b, j: (b, 0, 0)),
        ],
        out_specs=[
            pl.BlockSpec((1, 3 * BLK, 1), lambda b, j: (b * NB + j, 0, 0)),
            pl.BlockSpec((1, 3 * BLK, 16), lambda b, j: (b * NB + j, 0, 0)),
        ],
        out_shape=[
            jax.ShapeDtypeStruct((NBLK, 3 * BLK, 1), jnp.int32),
            jax.ShapeDtypeStruct((NBLK, 3 * BLK, 16), jnp.float32),
        ],
    )(xyz1t, xyz2t)


def _sc_body(table_hbm, gidx_hbm, wexp_hbm, fused_hbm,
             gidx_v, w_v, rows0, rows1, rows2, out_v, sem):
    wid = lax.axis_index("s") * 2 + lax.axis_index("c")
    rows = (rows0, rows1, rows2)
    for half in range(BLK_PER_W):
        blk = wid * BLK_PER_W + half
        pltpu.sync_copy(gidx_hbm.at[blk], gidx_v)
        pltpu.sync_copy(wexp_hbm.at[blk], w_v)
        for t in range(BLK // CHQ):
            cps = [
                pltpu.async_copy(
                    table_hbm.at[gidx_v.at[pl.ds(k * BLK + t * CHQ, CHQ)]],
                    rows[k], sem)
                for k in range(3)
            ]
            for cp in cps:
                cp.wait()

            def body(q, _):
                qg = t * CHQ + q
                w0 = w_v[pl.ds(qg * 16, 16)]
                w1 = w_v[pl.ds((BLK + qg) * 16, 16)]
                w2 = w_v[pl.ds((2 * BLK + qg) * 16, 16)]
                for cb in range(C // 16):
                    sl = pl.ds(cb * 16, 16)
                    out_v[q, sl] = (w0 * rows0[q, sl]
                                    + w1 * rows1[q, sl]) + w2 * rows2[q, sl]
                return 0

            lax.fori_loop(0, CHQ, body, 0)
            pltpu.sync_copy(out_v, fused_hbm.at[blk, pl.ds(t * CHQ, CHQ)])


def _sc_gather(table, gidx, wexp):
    kern = pl.kernel(
        _sc_body,
        out_type=jax.ShapeDtypeStruct((NBLK, BLK, C), jnp.float32),
        mesh=plsc.VectorSubcoreMesh(core_axis_name="c", subcore_axis_name="s",
                                    num_cores=2, num_subcores=16),
        scratch_types=[
            pltpu.VMEM((3 * BLK,), jnp.int32),
            pltpu.VMEM((3 * BLK * 16,), jnp.float32),
            pltpu.VMEM((CHQ, C), jnp.float32),
            pltpu.VMEM((CHQ, C), jnp.float32),
            pltpu.VMEM((CHQ, C), jnp.float32),
            pltpu.VMEM((CHQ, C), jnp.float32),
            pltpu.SemaphoreType.DMA,
        ],
    )
    return kern(table, gidx.reshape(NBLK, 3 * BLK),
                wexp.reshape(NBLK, 3 * BLK * 16))


def _accum_stats(h, s_ref, q_ref):
    s_blk = jnp.sum(h, axis=1, keepdims=True)
    q_blk = jnp.sum(h * h, axis=1, keepdims=True)
    first = (pl.program_id(0) == 0) & (pl.program_id(1) == 0)

    @pl.when(first)
    def _():
        s_ref[...] = s_blk
        q_ref[...] = q_blk

    @pl.when(jnp.logical_not(first))
    def _():
        s_ref[...] = s_ref[...] + s_blk
        q_ref[...] = q_ref[...] + q_blk


def _bn_affine(s_ref, q_ref, g_ref, be_ref):
    mean = s_ref[...] * (1.0 / NTOT)
    var = q_ref[...] * (1.0 / NTOT) - mean * mean
    a = g_ref[...] / jnp.sqrt(var + EPS)
    c = be_ref[...] - mean * a
    return a, c


def _k2_body(p1_ref, f_ref, pb_ref, w_ref, b_ref, h_ref, s_ref, q_ref):
    w = w_ref[...]
    h = lax.dot_general(w[:, :C], p1_ref[0], (((1,), (0,)), ((), ())),
                        preferred_element_type=jnp.float32)
    h = h + lax.dot_general(w[:, C:2 * C], f_ref[0], (((1,), (1,)), ((), ())),
                            preferred_element_type=jnp.float32)
    h = h + lax.dot_general(w[:, 2 * C:], pb_ref[0], (((1,), (0,)), ((), ())),
                            preferred_element_type=jnp.float32)
    h = h + b_ref[...]
    h_ref[0] = h.astype(jnp.bfloat16)
    _accum_stats(h, s_ref, q_ref)


def _k2(points1, fused, points_b1, W0, b0c):
    co = W0.shape[0]
    return pl.pallas_call(
        _k2_body,
        grid=(B, NBA),
        in_specs=[
            pl.BlockSpec((1, C, BLKA), lambda b, j: (b, 0, j)),
            pl.BlockSpec((1, BLKA, C), lambda b, j: (b, j, 0)),
            pl.BlockSpec((1, C, BLKA), lambda b, j: (b, 0, j)),
            pl.BlockSpec((co, 3 * C), lambda b, j: (0, 0)),
            pl.BlockSpec((co, 1), lambda b, j: (0, 0)),
        ],
        out_specs=[
            pl.BlockSpec((1, co, BLKA), lambda b, j: (b, 0, j)),
            pl.BlockSpec((co, 1), lambda b, j: (0, 0)),
            pl.BlockSpec((co, 1), lambda b, j: (0, 0)),
        ],
        out_shape=[
            jax.ShapeDtypeStruct((B, co, N1), jnp.bfloat16),
            jax.ShapeDtypeStruct((co, 1), jnp.float32),
            jax.ShapeDtypeStruct((co, 1), jnp.float32),
        ],
    )(points1, fused, points_b1, W0, b0c)


def _k3_body(h0_ref, s0_ref, q0_ref, g0_ref, be0_ref, w_ref, b_ref,
             h_ref, s_ref, q_ref):
    a, c = _bn_affine(s0_ref, q0_ref, g0_ref, be0_ref)
    xh = jnp.maximum(h0_ref[0].astype(jnp.float32) * a + c, 0.0)
    h = lax.dot_general(w_ref[...], xh, (((1,), (0,)), ((), ())),
                        preferred_element_type=jnp.float32)
    h = h + b_ref[...]
    h_ref[0] = h.astype(jnp.bfloat16)
    _accum_stats(h, s_ref, q_ref)


def _k3(h0, s0, q0, g0c, be0c, W1, b1c):
    ci, co = W1.shape[1], W1.shape[0]
    return pl.pallas_call(
        _k3_body,
        grid=(B, NBA),
        in_specs=[
            pl.BlockSpec((1, ci, BLKA), lambda b, j: (b, 0, j)),
            pl.BlockSpec((ci, 1), lambda b, j: (0, 0)),
            pl.BlockSpec((ci, 1), lambda b, j: (0, 0)),
            pl.BlockSpec((ci, 1), lambda b, j: (0, 0)),
            pl.BlockSpec((ci, 1), lambda b, j: (0, 0)),
            pl.BlockSpec((co, ci), lambda b, j: (0, 0)),
            pl.BlockSpec((co, 1), lambda b, j: (0, 0)),
        ],
        out_specs=[
            pl.BlockSpec((1, co, BLKA), lambda b, j: (b, 0, j)),
            pl.BlockSpec((co, 1), lambda b, j: (0, 0)),
            pl.BlockSpec((co, 1), lambda b, j: (0, 0)),
        ],
        out_shape=[
            jax.ShapeDtypeStruct((B, co, N1), jnp.bfloat16),
            jax.ShapeDtypeStruct((co, 1), jnp.float32),
            jax.ShapeDtypeStruct((co, 1), jnp.float32),
        ],
    )(h0, s0, q0, g0c, be0c, W1, b1c)


def _k4_body(h1_ref, s1_ref, q1_ref, g1_ref, be1_ref, o_ref):
    a, c = _bn_affine(s1_ref, q1_ref, g1_ref, be1_ref)
    y = h1_ref[0].astype(jnp.float32) * a + c
    o_ref[0, 0, :] = jnp.maximum(jnp.max(y, axis=0), 0.0)


def _k4(h1, s1, q1, g1c, be1c):
    ci = h1.shape[1]
    return pl.pallas_call(
        _k4_body,
        grid=(B, NBB),
        in_specs=[
            pl.BlockSpec((1, ci, BLKB), lambda b, j: (b, 0, j)),
            pl.BlockSpec((ci, 1), lambda b, j: (0, 0)),
            pl.BlockSpec((ci, 1), lambda b, j: (0, 0)),
            pl.BlockSpec((ci, 1), lambda b, j: (0, 0)),
            pl.BlockSpec((ci, 1), lambda b, j: (0, 0)),
        ],
        out_specs=pl.BlockSpec((1, 1, BLKB), lambda b, j: (b, 0, j)),
        out_shape=jax.ShapeDtypeStruct((B, 1, N1), jnp.float32),
    )(h1, s1, q1, g1c, be1c)


def kernel(xyz1, xyz2, points2, points1, points_b1,
           W0, b0, gamma0, beta0, W1, b1, gamma1, beta1):
    xyz1t = jnp.transpose(xyz1, (0, 2, 1))
    xyz2t = jnp.transpose(xyz2, (0, 2, 1))
    table = jnp.transpose(points2, (0, 2, 1)).reshape(B * N2, C)

    gidx, wexp = _k1(xyz1t, xyz2t)
    fused = _sc_gather(table, gidx, wexp).reshape(B, N1, C)

    h0, s0, q0 = _k2(points1, fused, points_b1, W0, b0[:, None])
    h1, s1, q1 = _k3(h0, s0, q0, gamma0[:, None], beta0[:, None],
                     W1, b1[:, None])
    out = _k4(h1, s1, q1, gamma1[:, None], beta1[:, None])
    return out.reshape(B, N1)


# speedup vs baseline: 19.8863x; 1.2207x over previous
import functools

import jax
import jax.numpy as jnp
from jax import lax
from jax.experimental import pallas as pl
from jax.experimental.pallas import tpu as pltpu
from jax.experimental.pallas import tpu_sc as plsc

B, N1, N2, C = 8, 4096, 1024, 128
EPS = 1e-5
BLK = 512
NB = N1 // BLK
NBLK = B * NB
CHQ = 128
NW = 32
BLK_PER_W = NBLK // NW
BLKA = 1024
NBA = N1 // BLKA
BLKB = 2048
NBB = N1 // BLKB
NTOT = B * N1


def _k1_body(x1_ref, x2_ref, gidx_ref, wq_ref):
    b = pl.program_id(0)
    d = None
    for c in range(3):
        t = x1_ref[0, c, :][:, None] - x2_ref[0, c, :][None, :]
        d = t * t if d is None else d + t * t
    it2 = lax.broadcasted_iota(jnp.int32, (N2, 2), 0)
    itcols = jnp.where(lax.broadcasted_iota(jnp.int32, (N2, 2), 1) == 0,
                       it2 >> 5, it2 & 31).astype(jnp.float32)
    inf = jnp.float32(jnp.inf)
    ws = []
    for k in range(3):
        m = jnp.min(d, axis=1, keepdims=True)
        onef = jnp.where(d == m, 1.0, 0.0)
        ikf = lax.dot_general(onef, itcols, (((1,), (0,)), ((), ())),
                              preferred_element_type=jnp.float32)
        iki = ikf.astype(jnp.int32)
        ik = jnp.minimum((iki[:, 0:1] << 5) + iki[:, 1:2], N2 - 1)
        gidx_ref[0, pl.ds(k * BLK, BLK), :] = ik + b * N2
        ws.append(1.0 / m)
        if k < 2:
            d = jnp.where(d == m, inf, d)
    s = (ws[0] + ws[1]) + ws[2]
    for k in range(3):
        wq_ref[0, pl.ds(k * BLK, BLK), :] = ws[k] / s


def _k1(xyz1t, xyz2t):
    return pl.pallas_call(
        _k1_body,
        grid=(B, NB),
        in_specs=[
            pl.BlockSpec((1, 3, BLK), lambda b, j: (b, 0, j)),
            pl.BlockSpec((1, 3, N2), lambda b, j: (b, 0, 0)),
        ],
        out_specs=[
            pl.BlockSpec((1, 3 * BLK, 1), lambda b, j: (b * NB + j, 0, 0)),
            pl.BlockSpec((1, 3 * BLK, 1), lambda b, j: (b * NB + j, 0, 0)),
        ],
        out_shape=[
            jax.ShapeDtypeStruct((NBLK, 3 * BLK, 1), jnp.int32),
            jax.ShapeDtypeStruct((NBLK, 3 * BLK, 1), jnp.float32),
        ],
    )(xyz1t, xyz2t)


def _sc_body(table_hbm, gidx_hbm, gath_hbm, gidx_v, rows0, rows1, rows2, sem):
    wid = lax.axis_index("s") * 2 + lax.axis_index("c")
    rows = (rows0, rows1, rows2)
    for half in range(BLK_PER_W):
        blk = wid * BLK_PER_W + half
        pltpu.sync_copy(gidx_hbm.at[blk], gidx_v)
        for t in range(BLK // CHQ):
            cps = [
                pltpu.async_copy(
                    table_hbm.at[gidx_v.at[pl.ds(k * BLK + t * CHQ, CHQ)]],
                    rows[k], sem)
                for k in range(3)
            ]
            for cp in cps:
                cp.wait()
            for k in range(3):
                pltpu.sync_copy(
                    rows[k], gath_hbm.at[blk, k, pl.ds(t * CHQ, CHQ)])


def _sc_gather(table, gidx):
    kern = pl.kernel(
        _sc_body,
        out_type=jax.ShapeDtypeStruct((NBLK, 3, BLK, C), jnp.float32),
        mesh=plsc.VectorSubcoreMesh(core_axis_name="c", subcore_axis_name="s",
                                    num_cores=2, num_subcores=16),
        scratch_types=[
            pltpu.VMEM((3 * BLK,), jnp.int32),
            pltpu.VMEM((CHQ, C), jnp.float32),
            pltpu.VMEM((CHQ, C), jnp.float32),
            pltpu.VMEM((CHQ, C), jnp.float32),
            pltpu.SemaphoreType.DMA,
        ],
    )
    return kern(table, gidx.reshape(NBLK, 3 * BLK))


def _accum_stats(h, s_ref, q_ref):
    s_blk = jnp.sum(h, axis=1, keepdims=True)
    q_blk = jnp.sum(h * h, axis=1, keepdims=True)
    first = (pl.program_id(0) == 0) & (pl.program_id(1) == 0)

    @pl.when(first)
    def _():
        s_ref[...] = s_blk
        q_ref[...] = q_blk

    @pl.when(jnp.logical_not(first))
    def _():
        s_ref[...] = s_ref[...] + s_blk
        q_ref[...] = q_ref[...] + q_blk


def _bn_affine(s_ref, q_ref, g_ref, be_ref):
    mean = s_ref[...] * (1.0 / NTOT)
    var = q_ref[...] * (1.0 / NTOT) - mean * mean
    a = g_ref[...] / jnp.sqrt(var + EPS)
    c = be_ref[...] - mean * a
    return a, c


def _k2_body(p1_ref, g_ref, wq_ref, pb_ref, w_ref, b_ref, h_ref, s_ref, q_ref):
    fs = []
    for u in range(BLKA // BLK):
        wk = [wq_ref[u, pl.ds(k * BLK, BLK), :] for k in range(3)]
        fs.append((wk[0] * g_ref[u, 0] + wk[1] * g_ref[u, 1])
                  + wk[2] * g_ref[u, 2])
    fused = jnp.concatenate(fs, axis=0)
    w = w_ref[...]
    h = lax.dot_general(w[:, :C], p1_ref[0], (((1,), (0,)), ((), ())),
                        preferred_element_type=jnp.float32)
    h = h + lax.dot_general(w[:, C:2 * C], fused, (((1,), (1,)), ((), ())),
                            preferred_element_type=jnp.float32)
    h = h + lax.dot_general(w[:, 2 * C:], pb_ref[0], (((1,), (0,)), ((), ())),
                            preferred_element_type=jnp.float32)
    h = h + b_ref[...]
    h_ref[0] = h.astype(jnp.bfloat16)
    _accum_stats(h, s_ref, q_ref)


def _k2(points1, gath, wq, points_b1, W0, b0c):
    co = W0.shape[0]
    bpa = BLKA // BLK
    return pl.pallas_call(
        _k2_body,
        grid=(B, NBA),
        in_specs=[
            pl.BlockSpec((1, C, BLKA), lambda b, j: (b, 0, j)),
            pl.BlockSpec((bpa, 3, BLK, C), lambda b, j: (b * NBA + j, 0, 0, 0)),
            pl.BlockSpec((bpa, 3 * BLK, 1), lambda b, j: (b * NBA + j, 0, 0)),
            pl.BlockSpec((1, C, BLKA), lambda b, j: (b, 0, j)),
            pl.BlockSpec((co, 3 * C), lambda b, j: (0, 0)),
            pl.BlockSpec((co, 1), lambda b, j: (0, 0)),
        ],
        out_specs=[
            pl.BlockSpec((1, co, BLKA), lambda b, j: (b, 0, j)),
            pl.BlockSpec((co, 1), lambda b, j: (0, 0)),
            pl.BlockSpec((co, 1), lambda b, j: (0, 0)),
        ],
        out_shape=[
            jax.ShapeDtypeStruct((B, co, N1), jnp.bfloat16),
            jax.ShapeDtypeStruct((co, 1), jnp.float32),
            jax.ShapeDtypeStruct((co, 1), jnp.float32),
        ],
    )(points1, gath, wq, points_b1, W0, b0c)


def _k3_body(h0_ref, s0_ref, q0_ref, g0_ref, be0_ref, w_ref, b_ref,
             h_ref, s_ref, q_ref):
    a, c = _bn_affine(s0_ref, q0_ref, g0_ref, be0_ref)
    xh = jnp.maximum(h0_ref[0].astype(jnp.float32) * a + c, 0.0)
    h = lax.dot_general(w_ref[...], xh, (((1,), (0,)), ((), ())),
                        preferred_element_type=jnp.float32)
    h = h + b_ref[...]
    h_ref[0] = h.astype(jnp.bfloat16)
    _accum_stats(h, s_ref, q_ref)


def _k3(h0, s0, q0, g0c, be0c, W1, b1c):
    ci, co = W1.shape[1], W1.shape[0]
    return pl.pallas_call(
        _k3_body,
        grid=(B, NBA),
        in_specs=[
            pl.BlockSpec((1, ci, BLKA), lambda b, j: (b, 0, j)),
            pl.BlockSpec((ci, 1), lambda b, j: (0, 0)),
            pl.BlockSpec((ci, 1), lambda b, j: (0, 0)),
            pl.BlockSpec((ci, 1), lambda b, j: (0, 0)),
            pl.BlockSpec((ci, 1), lambda b, j: (0, 0)),
            pl.BlockSpec((co, ci), lambda b, j: (0, 0)),
            pl.BlockSpec((co, 1), lambda b, j: (0, 0)),
        ],
        out_specs=[
            pl.BlockSpec((1, co, BLKA), lambda b, j: (b, 0, j)),
            pl.BlockSpec((co, 1), lambda b, j: (0, 0)),
            pl.BlockSpec((co, 1), lambda b, j: (0, 0)),
        ],
        out_shape=[
            jax.ShapeDtypeStruct((B, co, N1), jnp.bfloat16),
            jax.ShapeDtypeStruct((co, 1), jnp.float32),
            jax.ShapeDtypeStruct((co, 1), jnp.float32),
        ],
    )(h0, s0, q0, g0c, be0c, W1, b1c)


def _k4_body(h1_ref, s1_ref, q1_ref, g1_ref, be1_ref, o_ref):
    a, c = _bn_affine(s1_ref, q1_ref, g1_ref, be1_ref)
    y = h1_ref[0].astype(jnp.float32) * a + c
    o_ref[0, 0, :] = jnp.maximum(jnp.max(y, axis=0), 0.0)


def _k4(h1, s1, q1, g1c, be1c):
    ci = h1.shape[1]
    return pl.pallas_call(
        _k4_body,
        grid=(B, NBB),
        in_specs=[
            pl.BlockSpec((1, ci, BLKB), lambda b, j: (b, 0, j)),
            pl.BlockSpec((ci, 1), lambda b, j: (0, 0)),
            pl.BlockSpec((ci, 1), lambda b, j: (0, 0)),
            pl.BlockSpec((ci, 1), lambda b, j: (0, 0)),
            pl.BlockSpec((ci, 1), lambda b, j: (0, 0)),
        ],
        out_specs=pl.BlockSpec((1, 1, BLKB), lambda b, j: (b, 0, j)),
        out_shape=jax.ShapeDtypeStruct((B, 1, N1), jnp.float32),
    )(h1, s1, q1, g1c, be1c)


def kernel(xyz1, xyz2, points2, points1, points_b1,
           W0, b0, gamma0, beta0, W1, b1, gamma1, beta1):
    xyz1t = jnp.transpose(xyz1, (0, 2, 1))
    xyz2t = jnp.transpose(xyz2, (0, 2, 1))
    table = jnp.transpose(points2, (0, 2, 1)).reshape(B * N2, C)

    gidx, wq = _k1(xyz1t, xyz2t)
    gath = _sc_gather(table, gidx)

    h0, s0, q0 = _k2(points1, gath, wq, points_b1, W0, b0[:, None])
    h1, s1, q1 = _k3(h0, s0, q0, gamma0[:, None], beta0[:, None],
                     W1, b1[:, None])
    out = _k4(h1, s1, q1, gamma1[:, None], beta1[:, None])
    return out.reshape(B, N1)
